# Initial kernel scaffold; baseline (speedup 1.0000x reference)
import jax
import jax.numpy as jnp
from jax.experimental import pallas as pl


def kernel(x, edge_index, W, b):
    raise NotImplementedError("write your pallas kernel here")


# speedup vs baseline: 36.8378x; 36.8378x over previous
import functools

import jax
import jax.numpy as jnp
from jax import lax
from jax.experimental import pallas as pl
from jax.experimental.pallas import tpu as pltpu
from jax.experimental.pallas import tpu_sc as plsc

NC = 2
NS = 16
NW = NC * NS
CH = 128
F = 16


def _sc_mesh():
    return plsc.VectorSubcoreMesh(core_axis_name="c", subcore_axis_name="s")


def _make_deg_kernel(nchunk, npad):
    rows_t = npad // NS

    @functools.partial(
        pl.kernel,
        out_type=jax.ShapeDtypeStruct((NC, npad, F), jnp.float32),
        mesh=_sc_mesh(),
        scratch_types=[
            pltpu.VMEM((nchunk, CH), jnp.int32),
            pltpu.VMEM((CH, F), jnp.float32),
            pltpu.VMEM_SHARED((npad, F), jnp.float32),
        ],
        compiler_params=pltpu.CompilerParams(use_tc_tiling_on_sc=False),
    )
    def deg_kernel(dst_hbm, zeros_hbm, ones_hbm, out_hbm, idx_v, ones_v, acc_sh):
        c = lax.axis_index("c")
        s = lax.axis_index("s")
        wid = c * NS + s
        pltpu.sync_copy(dst_hbm.at[wid], idx_v)
        pltpu.sync_copy(ones_hbm, ones_v)
        pltpu.sync_copy(zeros_hbm, acc_sh.at[pl.ds(s * rows_t, rows_t)])
        plsc.subcore_barrier()

        @pl.loop(0, nchunk)
        def _(j):
            pltpu.sync_copy(ones_v, acc_sh.at[idx_v.at[j]], add=True)

        plsc.subcore_barrier()
        pltpu.sync_copy(acc_sh.at[pl.ds(s * rows_t, rows_t)],
                        out_hbm.at[c, pl.ds(s * rows_t, rows_t)])

    return deg_kernel


def _make_scatter_kernel(nchunk, npad, n):
    rows_t = npad // NS

    @functools.partial(
        pl.kernel,
        out_type=jax.ShapeDtypeStruct((NC, npad, F), jnp.float32),
        mesh=_sc_mesh(),
        scratch_types=[
            pltpu.VMEM((nchunk, CH), jnp.int32),
            pltpu.VMEM((nchunk, CH), jnp.int32),
            pltpu.VMEM((CH, F), jnp.float32),
            pltpu.VMEM_SHARED((npad, F), jnp.float32),
            pltpu.SemaphoreType.DMA,
        ],
        compiler_params=pltpu.CompilerParams(use_tc_tiling_on_sc=False),
    )
    def scatter_kernel(src_hbm, dst_hbm, g_hbm, zeros_hbm, out_hbm,
                       sidx_v, didx_v, rows_v, acc_sh, sem):
        c = lax.axis_index("c")
        s = lax.axis_index("s")
        wid = c * NS + s
        pltpu.sync_copy(src_hbm.at[wid], sidx_v)
        pltpu.sync_copy(dst_hbm.at[wid], didx_v)
        pltpu.sync_copy(zeros_hbm, acc_sh.at[pl.ds(s * rows_t, rows_t)])
        plsc.subcore_barrier()

        @pl.loop(0, nchunk)
        def _(j):
            pltpu.async_copy(g_hbm.at[sidx_v.at[j]], rows_v, sem).wait()
            pltpu.sync_copy(rows_v, acc_sh.at[didx_v.at[j]], add=True)

        plsc.subcore_barrier()
        pltpu.sync_copy(acc_sh.at[pl.ds(s * rows_t, rows_t)],
                        out_hbm.at[c, pl.ds(s * rows_t, rows_t)])

    return scatter_kernel


def _mid_body(x_ref, w_ref, degp_ref, g_ref, dinv_ref):
    h = jnp.dot(x_ref[...], w_ref[...], preferred_element_type=jnp.float32)
    deg = degp_ref[0] + degp_ref[1] + 1.0
    dinv = lax.rsqrt(deg)
    n = x_ref.shape[0]
    dinv_n = dinv[:n]
    g_ref[...] = h * dinv_n
    dinv_ref[...] = dinv_n


def _final_body(accp_ref, g_ref, dinv_ref, b_ref, out_ref):
    n = g_ref.shape[0]
    acc = accp_ref[0, :n] + accp_ref[1, :n]
    out_ref[...] = (acc + g_ref[...]) * dinv_ref[...] + b_ref[...]


def kernel(x, edge_index, W, b):
    n, f_in = x.shape
    f_out = W.shape[1]
    e = edge_index.shape[1]
    assert f_out == F

    npad = ((n + 1 + NS * F - 1) // (NS * F)) * (NS * F)
    per_tile = -(-e // (NW * CH * 8)) * CH * 8
    nchunk = per_tile // CH
    epad = per_tile * NW

    src = edge_index[0]
    dst = edge_index[1]
    pad = epad - e
    src_p = jnp.concatenate([src, jnp.zeros((pad,), jnp.int32)])
    dst_p = jnp.concatenate([dst, jnp.full((pad,), npad - 1, jnp.int32)])
    src_r = src_p.reshape(NW, nchunk, CH)
    dst_r = dst_p.reshape(NW, nchunk, CH)

    rows_t = npad // NS
    zeros_h = jnp.zeros((rows_t, F), jnp.float32)
    ones_h = jnp.ones((CH, F), jnp.float32)
    degp = _make_deg_kernel(nchunk, npad)(dst_r, zeros_h, ones_h)

    g, dinv = pl.pallas_call(
        _mid_body,
        out_shape=(
            jax.ShapeDtypeStruct((n, F), jnp.float32),
            jax.ShapeDtypeStruct((n, F), jnp.float32),
        ),
    )(x, W, degp)

    accp = _make_scatter_kernel(nchunk, npad, n)(src_r, dst_r, g, zeros_h)

    out = pl.pallas_call(
        _final_body,
        out_shape=jax.ShapeDtypeStruct((n, F), jnp.float32),
    )(accp, g, dinv, b.reshape(1, F))
    return out


# speedup vs baseline: 45.6863x; 1.2402x over previous
import functools

import jax
import jax.numpy as jnp
from jax import lax
from jax.experimental import pallas as pl
from jax.experimental.pallas import tpu as pltpu
from jax.experimental.pallas import tpu_sc as plsc

NC = 2
NS = 16
NW = NC * NS
CH = 128
KB = 8
F = 16

_SC_PARAMS = pltpu.CompilerParams(use_tc_tiling_on_sc=False)


def _sc_mesh():
    return plsc.VectorSubcoreMesh(core_axis_name="c", subcore_axis_name="s")


def _make_deg_kernel(nchunk, npad):
    rows_t = npad // NS

    @functools.partial(
        pl.kernel,
        out_type=jax.ShapeDtypeStruct((NC, npad), jnp.float32),
        mesh=_sc_mesh(),
        scratch_types=[
            pltpu.VMEM((nchunk, CH), jnp.int32),
            pltpu.VMEM((CH,), jnp.float32),
            pltpu.VMEM_SHARED((npad,), jnp.float32),
            pltpu.SemaphoreType.DMA,
        ],
        compiler_params=_SC_PARAMS,
    )
    def deg_kernel(dst_hbm, zeros_hbm, ones_hbm, out_hbm, idx_v, ones_v,
                   acc_sh, sem):
        c = lax.axis_index("c")
        s = lax.axis_index("s")
        wid = c * NS + s
        pltpu.sync_copy(dst_hbm.at[wid], idx_v)
        pltpu.sync_copy(ones_hbm, ones_v)
        pltpu.sync_copy(zeros_hbm, acc_sh.at[pl.ds(s * rows_t, rows_t)])
        plsc.subcore_barrier()

        @pl.loop(0, nchunk)
        def _(j):
            pltpu.async_copy(ones_v, acc_sh.at[idx_v.at[j]], sem, add=True)

        @pl.loop(0, nchunk)
        def _(j):
            pltpu.make_async_copy(ones_v, acc_sh.at[idx_v.at[j]], sem).wait()

        plsc.subcore_barrier()
        pltpu.sync_copy(acc_sh.at[pl.ds(s * rows_t, rows_t)],
                        out_hbm.at[c, pl.ds(s * rows_t, rows_t)])

    return deg_kernel


def _make_scatter_kernel(nchunk, npad):
    rows_t = npad // NS
    assert nchunk % KB == 0

    @functools.partial(
        pl.kernel,
        out_type=jax.ShapeDtypeStruct((NC, npad, F), jnp.float32),
        mesh=_sc_mesh(),
        scratch_types=[
            pltpu.VMEM((nchunk, CH), jnp.int32),
            pltpu.VMEM((nchunk, CH), jnp.int32),
            pltpu.VMEM((KB, CH, F), jnp.float32),
            pltpu.VMEM_SHARED((npad, F), jnp.float32),
            pltpu.SemaphoreType.DMA,
            pltpu.SemaphoreType.DMA,
        ],
        compiler_params=_SC_PARAMS,
    )
    def scatter_kernel(src_hbm, dst_hbm, g_hbm, zeros_hbm, out_hbm,
                       sidx_v, didx_v, rows_v, acc_sh, gsem, ssem):
        c = lax.axis_index("c")
        s = lax.axis_index("s")
        wid = c * NS + s
        pltpu.sync_copy(src_hbm.at[wid], sidx_v)
        pltpu.sync_copy(dst_hbm.at[wid], didx_v)
        pltpu.sync_copy(zeros_hbm, acc_sh.at[pl.ds(s * rows_t, rows_t)])
        plsc.subcore_barrier()

        @pl.loop(0, nchunk, step=KB)
        def _(j0):
            for q in range(KB):
                pltpu.async_copy(
                    g_hbm.at[sidx_v.at[j0 + q]], rows_v.at[q], gsem)
            for q in range(KB):
                pltpu.make_async_copy(
                    g_hbm.at[sidx_v.at[j0 + q]], rows_v.at[q], gsem).wait()
            for q in range(KB):
                pltpu.async_copy(
                    rows_v.at[q], acc_sh.at[didx_v.at[j0 + q]], ssem,
                    add=True)
            for q in range(KB):
                pltpu.make_async_copy(
                    rows_v.at[q], acc_sh.at[didx_v.at[j0 + q]], ssem).wait()

        plsc.subcore_barrier()
        pltpu.sync_copy(acc_sh.at[pl.ds(s * rows_t, rows_t)],
                        out_hbm.at[c, pl.ds(s * rows_t, rows_t)])

    return scatter_kernel


def _mid_body(x_ref, w_ref, degp_ref, g_ref, dinv_ref):
    h = jnp.dot(x_ref[...], w_ref[...], preferred_element_type=jnp.float32)
    n = x_ref.shape[0]
    deg = degp_ref[0] + degp_ref[1] + 1.0
    dinv = lax.rsqrt(deg)[:n][:, None]
    g_ref[...] = h * dinv
    dinv_ref[...] = jnp.broadcast_to(dinv, g_ref.shape)


def _final_body(accp_ref, g_ref, dinv_ref, b_ref, out_ref):
    n = g_ref.shape[0]
    acc = accp_ref[0, :n] + accp_ref[1, :n]
    out_ref[...] = (acc + g_ref[...]) * dinv_ref[...] + b_ref[...]


def kernel(x, edge_index, W, b):
    n, f_in = x.shape
    f_out = W.shape[1]
    e = edge_index.shape[1]
    assert f_out == F

    npad = ((n + 1 + NS * F - 1) // (NS * F)) * (NS * F)
    per_tile = -(-e // (NW * CH * KB)) * CH * KB
    nchunk = per_tile // CH
    epad = per_tile * NW

    src = edge_index[0]
    dst = edge_index[1]
    pad = epad - e
    src_p = jnp.concatenate([src, jnp.zeros((pad,), jnp.int32)])
    dst_p = jnp.concatenate([dst, jnp.full((pad,), npad - 1, jnp.int32)])
    src_r = src_p.reshape(NW, nchunk, CH)
    dst_r = dst_p.reshape(NW, nchunk, CH)

    rows_t = npad // NS
    zeros1_h = jnp.zeros((rows_t,), jnp.float32)
    zeros2_h = jnp.zeros((rows_t, F), jnp.float32)
    ones_h = jnp.ones((CH,), jnp.float32)
    degp = _make_deg_kernel(nchunk, npad)(dst_r, zeros1_h, ones_h)

    g, dinv = pl.pallas_call(
        _mid_body,
        out_shape=(
            jax.ShapeDtypeStruct((n, F), jnp.float32),
            jax.ShapeDtypeStruct((n, F), jnp.float32),
        ),
    )(x, W, degp)

    accp = _make_scatter_kernel(nchunk, npad)(src_r, dst_r, g, zeros2_h)

    out = pl.pallas_call(
        _final_body,
        out_shape=jax.ShapeDtypeStruct((n, F), jnp.float32),
    )(accp, g, dinv, b.reshape(1, F))
    return out
